# baseline (speedup 1.0000x reference)
import jax
import jax.numpy as jnp
from jax.experimental import pallas as pl
from jax.experimental.pallas import tpu as pltpu

_SPLIT = 8
_K = 16


def kernel(r_idx, qkv):
    n, p3, w3, c = qkv.shape
    topk = r_idx.shape[-1]
    rows = n * p3
    out_rows = rows * topk

    table = qkv.reshape(rows * _SPLIT, w3 // _SPLIT, c)
    gidx = (
        r_idx.astype(jnp.int32)
        + (jnp.arange(n, dtype=jnp.int32) * p3)[:, None, None]
    ).reshape(-1)

    def body(gidx_ref, table_any, out_any, sems):
        def cp(i, j):
            return pltpu.make_async_copy(
                table_any.at[pl.ds(gidx_ref[i] * _SPLIT, _SPLIT)],
                out_any.at[pl.ds(j * _SPLIT, _SPLIT)],
                sems.at[j % _K])

        def step(i, carry):
            @pl.when(i >= _K)
            def _():
                cp(i - _K, i - _K).wait()
            cp(i, i).start()
            return carry

        jax.lax.fori_loop(0, out_rows, step, 0)

        def drain(i, carry):
            cp(i, i).wait()
            return carry

        jax.lax.fori_loop(out_rows - _K, out_rows, drain, 0)

    grid_spec = pltpu.PrefetchScalarGridSpec(
        num_scalar_prefetch=1,
        grid=(1,),
        in_specs=[pl.BlockSpec(memory_space=pl.ANY)],
        out_specs=pl.BlockSpec(memory_space=pl.ANY),
        scratch_shapes=[pltpu.SemaphoreType.DMA((_K,))],
    )
    out = pl.pallas_call(
        body,
        grid_spec=grid_spec,
        out_shape=jax.ShapeDtypeStruct((out_rows * _SPLIT, w3 // _SPLIT, c),
                                       qkv.dtype),
    )(gidx, table)
    return out.reshape(n, p3, topk, w3, c)


# speedup vs baseline: 36.9897x; 36.9897x over previous
import functools

import jax
import jax.numpy as jnp
from jax import lax
from jax.experimental import pallas as pl
from jax.experimental.pallas import tpu as pltpu
from jax.experimental.pallas import tpu_sc as plsc

_NC, _NS = 2, 16
_NW = _NC * _NS
_SPLIT = 8
_CH = 16
_NBUF = 2


def kernel(r_idx, qkv):
    n, p3, w3, c = qkv.shape
    topk = r_idx.shape[-1]
    rows = n * p3
    out_rows = rows * topk
    d = w3 * c
    sd = d // _SPLIT
    rows_pt = out_rows // _NW
    sub_pt = rows_pt * _SPLIT
    chunks = rows_pt

    table = qkv.reshape(rows * _SPLIT, w3 // _SPLIT, c)

    rif = r_idx.astype(jnp.int32).reshape(-1)
    gidx = rif + (jnp.arange(out_rows, dtype=jnp.int32) // (p3 * topk)) * p3
    sidx = (gidx[:, None] * _SPLIT + jnp.arange(_SPLIT, dtype=jnp.int32)).reshape(
        _NW, sub_pt)
    sidx1d = jnp.pad(sidx, ((0, 0), (0, 512 - sub_pt))).reshape(-1)

    mesh = plsc.VectorSubcoreMesh(
        core_axis_name="c", subcore_axis_name="s",
        num_cores=_NC, num_subcores=_NS,
    )

    @functools.partial(
        pl.kernel,
        out_type=jax.ShapeDtypeStruct(
            (out_rows * _SPLIT, w3 // _SPLIT, c), jnp.float32),
        mesh=mesh,
        compiler_params=pltpu.CompilerParams(use_tc_tiling_on_sc=True),
        scratch_types=[
            pltpu.VMEM((512,), jnp.int32),
            pltpu.VMEM((_NBUF, _CH, w3 // _SPLIT, c), jnp.float32),
            [pltpu.SemaphoreType.DMA] * (2 * _NBUF),
            [pltpu.SemaphoreType.DMA] * (2 * _NBUF),
        ],
    )
    def sc_gather(sidx_hbm, table_hbm, out_hbm, sidx_v, buf, gsems, ssems):
        wid = lax.axis_index("s") * _NC + lax.axis_index("c")
        pltpu.sync_copy(sidx_hbm.at[pl.ds(wid * 512, 512)], sidx_v)

        out_base = wid * sub_pt
        spans = [(s0, min(_CH, sub_pt - s0)) for s0 in range(0, sub_pt, _CH)]
        nch = len(spans)

        def halves(s0, sz):
            h = max(sz // 2 // _SPLIT * _SPLIT, _SPLIT)
            if h >= sz:
                return [(s0, sz)]
            return [(s0, h), (s0 + h, sz - h)]

        def issue_gather(s0, sz, b):
            return tuple(
                pltpu.async_copy(
                    table_hbm.at[sidx_v.at[pl.ds(h0, hsz)]],
                    buf.at[b, pl.ds(h0 - s0, hsz)],
                    gsems[2 * b + i])
                for i, (h0, hsz) in enumerate(halves(s0, sz)))

        def issue_scatter(s0, sz, b):
            return tuple(
                pltpu.async_copy(
                    buf.at[b, pl.ds(h0 - s0, hsz)],
                    out_hbm.at[pl.ds(out_base + h0, hsz)],
                    ssems[2 * b + i])
                for i, (h0, hsz) in enumerate(halves(s0, sz)))

        def wait_all(cps):
            for cp in cps:
                cp.wait()

        gath = [None] * _NBUF
        scat = [None] * _NBUF
        for g, (s0, sz) in enumerate(spans):
            b = g % _NBUF
            if g >= _NBUF:
                wait_all(scat[b])
            gath[b] = issue_gather(s0, sz, b)
            if g >= 1:
                pb = (g - 1) % _NBUF
                p0, psz = spans[g - 1]
                wait_all(gath[pb])
                scat[pb] = issue_scatter(p0, psz, pb)
        lb = (nch - 1) % _NBUF
        l0, lsz = spans[-1]
        wait_all(gath[lb])
        scat[lb] = issue_scatter(l0, lsz, lb)
        for cch in range(max(0, nch - _NBUF), nch):
            wait_all(scat[cch % _NBUF])

    out = sc_gather(sidx1d, table)
    return out.reshape(n, p3, topk, w3, c)
